# baseline (speedup 1.0000x reference)
import functools

import jax
import jax.numpy as jnp
from jax import lax
from jax.experimental import pallas as pl
from jax.experimental.pallas import tpu as pltpu
from jax.experimental.pallas import tpu_sc as plsc

B_TOTAL = 4 * 4096
DIM = 1024
NUM_WORKERS = 32
B_PER_W = B_TOTAL // NUM_WORKERS
CHUNK = 16
N_CHUNKS = B_PER_W // CHUNK
NBUF = 7

_mesh = plsc.VectorSubcoreMesh(core_axis_name="c", subcore_axis_name="s")


@functools.partial(
    pl.kernel,
    mesh=_mesh,
    out_type=jax.ShapeDtypeStruct((B_TOTAL, DIM), jnp.float32),
    scratch_types=[
        pltpu.VMEM((N_CHUNKS, CHUNK), jnp.int32),
        *[pltpu.VMEM((CHUNK, DIM), jnp.float32) for _ in range(NBUF)],
        *[pltpu.SemaphoreType.DMA for _ in range(2 * NBUF)],
    ],
)
def _emb_gather(idx_hbm, table_hbm, out_hbm, idx_v, *scratch):
    bufs = scratch[:NBUF]
    gsems = scratch[NBUF:2 * NBUF]
    osems = scratch[2 * NBUF:]

    wid = lax.axis_index("s") * 2 + lax.axis_index("c")
    base = wid * B_PER_W

    pltpu.sync_copy(idx_hbm.at[wid], idx_v)

    def out_slice(g):
        return out_hbm.at[pl.ds(base + g * CHUNK, CHUNK)]

    for g in range(NBUF):
        pltpu.async_copy(table_hbm.at[idx_v.at[g]], bufs[g], gsems[g])

    for g in range(N_CHUNKS):
        b = g % NBUF
        pltpu.make_async_copy(table_hbm.at[idx_v.at[g]], bufs[b],
                              gsems[b]).wait()
        pltpu.async_copy(bufs[b], out_slice(g), osems[b])
        if g + NBUF < N_CHUNKS:
            pltpu.make_async_copy(bufs[b], out_slice(g), osems[b]).wait()
            pltpu.async_copy(table_hbm.at[idx_v.at[g + NBUF]], bufs[b],
                             gsems[b])

    for g in range(max(0, N_CHUNKS - NBUF), N_CHUNKS):
        pltpu.make_async_copy(bufs[g % NBUF], out_slice(g),
                              osems[g % NBUF]).wait()


def kernel(x, token_emb):
    idx = x.reshape(NUM_WORKERS, N_CHUNKS, CHUNK).astype(jnp.int32)
    out = _emb_gather(idx, token_emb)
    return out.reshape(x.shape + (DIM,))


# speedup vs baseline: 1.0013x; 1.0013x over previous
import functools

import jax
import jax.numpy as jnp
from jax import lax
from jax.experimental import pallas as pl
from jax.experimental.pallas import tpu as pltpu
from jax.experimental.pallas import tpu_sc as plsc

B_ROWS = 4
B_COLS = 4096
B_TOTAL = B_ROWS * B_COLS
DIM = 1024
NUM_WORKERS = 32
B_PER_W = B_TOTAL // NUM_WORKERS
W_PER_ROW = B_COLS // B_PER_W
CHUNK = 32
N_CHUNKS = B_PER_W // CHUNK
NBUF = 3

_mesh = plsc.VectorSubcoreMesh(core_axis_name="c", subcore_axis_name="s")


@functools.partial(
    pl.kernel,
    mesh=_mesh,
    out_type=jax.ShapeDtypeStruct((B_TOTAL, DIM), jnp.float32),
    scratch_types=[
        pltpu.VMEM((B_PER_W,), jnp.int32),
        *[pltpu.VMEM((CHUNK, DIM), jnp.float32) for _ in range(NBUF)],
        *[pltpu.SemaphoreType.DMA for _ in range(2 * NBUF)],
    ],
)
def _emb_gather(idx_hbm, table_hbm, out_hbm, idx_v, *scratch):
    bufs = scratch[:NBUF]
    gsems = scratch[NBUF:2 * NBUF]
    osems = scratch[2 * NBUF:]

    wid = lax.axis_index("s") * 2 + lax.axis_index("c")
    base = wid * B_PER_W

    row = wid // W_PER_ROW
    col = (wid % W_PER_ROW) * B_PER_W
    pltpu.sync_copy(idx_hbm.at[row, pl.ds(col, B_PER_W)], idx_v)

    def idx_slice(g):
        return idx_v.at[pl.ds(g * CHUNK, CHUNK)]

    def out_slice(g):
        return out_hbm.at[pl.ds(base + g * CHUNK, CHUNK)]

    for g in range(NBUF):
        pltpu.async_copy(table_hbm.at[idx_slice(g)], bufs[g], gsems[g])

    for g in range(N_CHUNKS):
        b = g % NBUF
        pltpu.make_async_copy(table_hbm.at[idx_slice(g)], bufs[b],
                              gsems[b]).wait()
        pltpu.async_copy(bufs[b], out_slice(g), osems[b])
        if g + NBUF < N_CHUNKS:
            pltpu.make_async_copy(bufs[b], out_slice(g), osems[b]).wait()
            pltpu.async_copy(table_hbm.at[idx_slice(g + NBUF)], bufs[b],
                             gsems[b])

    for g in range(max(0, N_CHUNKS - NBUF), N_CHUNKS):
        pltpu.make_async_copy(bufs[g % NBUF], out_slice(g),
                              osems[g % NBUF]).wait()


def kernel(x, token_emb):
    out = _emb_gather(x.astype(jnp.int32), token_emb)
    return out.reshape(x.shape + (DIM,))
